# Initial kernel scaffold; baseline (speedup 1.0000x reference)
import jax
import jax.numpy as jnp
from jax.experimental import pallas as pl


def kernel(x, edge_index, batch, W1, b1, W2, b2, fc1_W, fc1_b, ln_g, ln_b, fc2_W, fc2_b):
    raise NotImplementedError("write your pallas kernel here")


# speedup vs baseline: 2.7525x; 2.7525x over previous
import functools

import jax
import jax.numpy as jnp
from jax.experimental import pallas as pl
from jax.experimental.pallas import tpu as pltpu

_N = 10000
_NG = 16
_D = 128


def _leaky(x):
    return jnp.where(x >= 0, x, 0.01 * x)


def _head_kernel(h_ref, batch_ref, fc1w_ref, fc1b_ref, lng_ref, lnb_ref,
                 fc2w_ref, fc2b_ref, out_ref, sum_ref, cnt_ref):
    i = pl.program_id(0)
    nblk = pl.num_programs(0)

    @pl.when(i == 0)
    def _init():
        sum_ref[...] = jnp.zeros_like(sum_ref)
        cnt_ref[...] = jnp.zeros_like(cnt_ref)

    h = h_ref[...]
    b = batch_ref[...]
    onehot = (b == jax.lax.broadcasted_iota(jnp.int32, (1, _NG), 1)).astype(jnp.float32)
    sum_ref[...] += jax.lax.dot_general(onehot, h, (((0,), (0,)), ((), ())),
                                        precision=jax.lax.Precision.HIGHEST,
                                        preferred_element_type=jnp.float32)
    cnt_ref[...] += jnp.sum(onehot, axis=0, keepdims=True)

    @pl.when(i == nblk - 1)
    def _head():
        pooled = sum_ref[...] / jnp.maximum(cnt_ref[...], 1.0).reshape(_NG, 1)
        z = _leaky(jnp.dot(pooled, fc1w_ref[...],
                           precision=jax.lax.Precision.HIGHEST,
                           preferred_element_type=jnp.float32) + fc1b_ref[...])
        mu = jnp.mean(z, axis=-1, keepdims=True)
        var = jnp.mean((z - mu) ** 2, axis=-1, keepdims=True)
        z = (z - mu) * jax.lax.rsqrt(var + 1e-5) * lng_ref[...] + lnb_ref[...]
        out_ref[...] = jnp.dot(z, fc2w_ref[...],
                               precision=jax.lax.Precision.HIGHEST,
                               preferred_element_type=jnp.float32) + fc2b_ref[...]


def _run_head(h, batch2d, fc1_W, fc1_b, ln_g, ln_b, fc2_W, fc2_b):
    n_pad = h.shape[0]
    blk = 2000
    grid = (n_pad // blk,)
    return pl.pallas_call(
        _head_kernel,
        grid=grid,
        in_specs=[
            pl.BlockSpec((blk, _D), lambda i: (i, 0)),
            pl.BlockSpec((blk, 1), lambda i: (i, 0)),
            pl.BlockSpec((_D, 128), lambda i: (0, 0)),
            pl.BlockSpec((128,), lambda i: (0,)),
            pl.BlockSpec((128,), lambda i: (0,)),
            pl.BlockSpec((128,), lambda i: (0,)),
            pl.BlockSpec((128, 1), lambda i: (0, 0)),
            pl.BlockSpec((1,), lambda i: (0,)),
        ],
        out_specs=pl.BlockSpec((_NG, 1), lambda i: (0, 0)),
        out_shape=jax.ShapeDtypeStruct((_NG, 1), jnp.float32),
        scratch_shapes=[
            pltpu.VMEM((_NG, _D), jnp.float32),
            pltpu.VMEM((1, _NG), jnp.float32),
        ],
    )(h, batch2d, fc1_W, fc1_b, ln_g, ln_b, fc2_W, fc2_b)


def kernel(x, edge_index, batch, W1, b1, W2, b2, fc1_W, fc1_b, ln_g, ln_b,
           fc2_W, fc2_b):
    src, dst = edge_index[0], edge_index[1]
    n = x.shape[0]

    deg = jax.ops.segment_sum(jnp.ones_like(dst, dtype=jnp.float32), dst,
                              num_segments=n) + 1.0
    dis = jax.lax.rsqrt(deg)

    def gcn(h_in, W, b):
        g = dis[:, None] * (h_in @ W)
        acc = jax.ops.segment_sum(jnp.take(g, src, axis=0), dst, num_segments=n)
        return _leaky(dis[:, None] * (acc + g) + b)

    h = gcn(x, W1, b1)
    h = gcn(h, W2, b2)

    batch2d = batch.astype(jnp.int32).reshape(n, 1)
    return _run_head(h, batch2d, fc1_W, fc1_b, ln_g, ln_b, fc2_W, fc2_b)


# speedup vs baseline: 13.1337x; 4.7716x over previous
import functools

import jax
import jax.numpy as jnp
from jax.experimental import pallas as pl
from jax.experimental.pallas import tpu as pltpu
from jax.experimental.pallas import tpu_sc as plsc

_N = 10000
_N_PAD = 10240
_D = 128
_HD = 64
_NG = 16
_E = 320000
_E_PAD = 327680
_NCH_D = 80
_NCH_M = 160
_RPT = _N_PAD // 16
_BLK = 1024
_HI = jax.lax.Precision.HIGHEST

_mesh = plsc.VectorSubcoreMesh(core_axis_name="c", subcore_axis_name="s")


def _leaky(x):
    return jnp.where(x >= 0, x, 0.01 * x)


def _sc_deg(dst_r, ones_rows, zrow):
    @functools.partial(
        pl.kernel,
        out_type=jax.ShapeDtypeStruct((2, _N_PAD, _HD), jnp.float32),
        mesh=_mesh,
        compiler_params=pltpu.CompilerParams(use_tc_tiling_on_sc=False),
        scratch_types=[
            pltpu.VMEM((_NCH_D, 128), jnp.int32),
            pltpu.VMEM((128, _HD), jnp.float32),
            pltpu.VMEM_SHARED((_N_PAD, _HD), jnp.float32),
        ],
    )
    def k(dst_hbm, ones_hbm, z_hbm, out_hbm, dst_v, ones_v, table):
        c = jax.lax.axis_index("c")
        s = jax.lax.axis_index("s")
        w = c * 16 + s
        pltpu.sync_copy(dst_hbm.at[w], dst_v)
        pltpu.sync_copy(z_hbm, ones_v)
        for t in range(_RPT // 128):
            pltpu.sync_copy(ones_v, table.at[pl.ds(s * _RPT + t * 128, 128)])
        pltpu.sync_copy(ones_hbm, ones_v)
        plsc.subcore_barrier()

        def body(j, carry):
            pltpu.sync_copy(ones_v, table.at[dst_v.at[j]], add=True)
            return carry

        jax.lax.fori_loop(0, _NCH_D, body, 0)
        plsc.subcore_barrier()
        pltpu.sync_copy(table.at[pl.ds(s * _RPT, _RPT)],
                        out_hbm.at[c].at[pl.ds(s * _RPT, _RPT)])

    return k(dst_r, ones_rows, zrow)


def _sc_msg(g, src_r, dst_r, zrow):
    @functools.partial(
        pl.kernel,
        out_type=jax.ShapeDtypeStruct((2, _N_PAD, _HD), jnp.float32),
        mesh=_mesh,
        compiler_params=pltpu.CompilerParams(use_tc_tiling_on_sc=False),
        scratch_types=[
            pltpu.VMEM((_NCH_M, 128), jnp.int32),
            pltpu.VMEM((_NCH_M, 128), jnp.int32),
            pltpu.VMEM((128, _HD), jnp.float32),
            pltpu.VMEM((128, _HD), jnp.float32),
            pltpu.VMEM_SHARED((_N_PAD, _HD), jnp.float32),
            pltpu.SemaphoreType.DMA,
            pltpu.SemaphoreType.DMA,
        ],
    )
    def k(g_hbm, src_hbm, dst_hbm, z_hbm, out_hbm,
          src_v, dst_v, buf0, buf1, acc, sem0, sem1):
        c = jax.lax.axis_index("c")
        s = jax.lax.axis_index("s")
        pltpu.sync_copy(src_hbm.at[s], src_v)
        pltpu.sync_copy(dst_hbm.at[s], dst_v)
        pltpu.sync_copy(z_hbm, buf0)
        for t in range(_RPT // 128):
            pltpu.sync_copy(buf0, acc.at[pl.ds(s * _RPT + t * 128, 128)])
        plsc.subcore_barrier()

        pltpu.async_copy(g_hbm.at[c].at[src_v.at[0]], buf0, sem0)
        pltpu.async_copy(g_hbm.at[c].at[src_v.at[1]], buf1, sem1)

        def body(i, carry):
            j = 2 * i
            pltpu.make_async_copy(g_hbm.at[c].at[src_v.at[j]],
                                  buf0, sem0).wait()
            pltpu.sync_copy(buf0, acc.at[dst_v.at[j]], add=True)

            @pl.when(j + 2 < _NCH_M)
            def _():
                pltpu.async_copy(g_hbm.at[c].at[src_v.at[j + 2]], buf0, sem0)

            pltpu.make_async_copy(g_hbm.at[c].at[src_v.at[j + 1]],
                                  buf1, sem1).wait()
            pltpu.sync_copy(buf1, acc.at[dst_v.at[j + 1]], add=True)

            @pl.when(j + 3 < _NCH_M)
            def _():
                pltpu.async_copy(g_hbm.at[c].at[src_v.at[j + 3]], buf1, sem1)

            return carry

        jax.lax.fori_loop(0, _NCH_M // 2, body, 0)
        plsc.subcore_barrier()
        pltpu.sync_copy(acc.at[pl.ds(s * _RPT, _RPT)],
                        out_hbm.at[c].at[pl.ds(s * _RPT, _RPT)])

    return k(g, src_r, dst_r, zrow)


def _tc1_kernel(x_ref, w_ref, deg_ref, dis_ref, g_ref):
    deg = deg_ref[0][:, :1] + deg_ref[1][:, :1]
    dis = jax.lax.rsqrt(deg + 1.0)
    dis_ref[...] = dis
    g = jnp.dot(x_ref[...], w_ref[...], precision=_HI,
                preferred_element_type=jnp.float32) * dis
    g_ref[0] = g[:, :_HD]
    g_ref[1] = g[:, _HD:]


def _tc1(x_p, W1, deg2):
    return pl.pallas_call(
        _tc1_kernel,
        grid=(_N_PAD // _BLK,),
        in_specs=[
            pl.BlockSpec((_BLK, _D), lambda i: (i, 0)),
            pl.BlockSpec((_D, _D), lambda i: (0, 0)),
            pl.BlockSpec((2, _BLK, _HD), lambda i: (0, i, 0)),
        ],
        out_specs=[
            pl.BlockSpec((_BLK, 1), lambda i: (i, 0)),
            pl.BlockSpec((2, _BLK, _HD), lambda i: (0, i, 0)),
        ],
        out_shape=[
            jax.ShapeDtypeStruct((_N_PAD, 1), jnp.float32),
            jax.ShapeDtypeStruct((2, _N_PAD, _HD), jnp.float32),
        ],
    )(x_p, W1, deg2)


def _tc2_kernel(acc_ref, g_ref, dis_ref, b_ref, w_ref, out_ref):
    a = jnp.concatenate([acc_ref[0] + g_ref[0], acc_ref[1] + g_ref[1]],
                        axis=1)
    h = _leaky(a * dis_ref[...] + b_ref[...])
    o = jnp.dot(h, w_ref[...], precision=_HI,
                preferred_element_type=jnp.float32) * dis_ref[...]
    out_ref[0] = o[:, :_HD]
    out_ref[1] = o[:, _HD:]


def _tc2(acc1, g1, dis, b1, W2):
    return pl.pallas_call(
        _tc2_kernel,
        grid=(_N_PAD // _BLK,),
        in_specs=[
            pl.BlockSpec((2, _BLK, _HD), lambda i: (0, i, 0)),
            pl.BlockSpec((2, _BLK, _HD), lambda i: (0, i, 0)),
            pl.BlockSpec((_BLK, 1), lambda i: (i, 0)),
            pl.BlockSpec((_D,), lambda i: (0,)),
            pl.BlockSpec((_D, _D), lambda i: (0, 0)),
        ],
        out_specs=pl.BlockSpec((2, _BLK, _HD), lambda i: (0, i, 0)),
        out_shape=jax.ShapeDtypeStruct((2, _N_PAD, _HD), jnp.float32),
    )(acc1, g1, dis, b1, W2)


def _tc3_kernel(acc_ref, g_ref, dis_ref, b2_ref, batch_ref, fc1w_ref,
                fc1b_ref, lng_ref, lnb_ref, fc2w_ref, fc2b_ref, out_ref,
                sum_ref, cnt_ref):
    i = pl.program_id(0)
    nblk = pl.num_programs(0)

    @pl.when(i == 0)
    def _init():
        sum_ref[...] = jnp.zeros_like(sum_ref)
        cnt_ref[...] = jnp.zeros_like(cnt_ref)

    a = jnp.concatenate([acc_ref[0] + g_ref[0], acc_ref[1] + g_ref[1]],
                        axis=1)
    h = _leaky(a * dis_ref[...] + b2_ref[...])
    b = batch_ref[...]
    onehot = (b == jax.lax.broadcasted_iota(jnp.int32, (1, _NG), 1))
    onehot = onehot.astype(jnp.float32)
    sum_ref[...] += jax.lax.dot_general(onehot, h, (((0,), (0,)), ((), ())),
                                        precision=_HI,
                                        preferred_element_type=jnp.float32)
    cnt_ref[...] += jnp.sum(onehot, axis=0, keepdims=True)

    @pl.when(i == nblk - 1)
    def _head():
        pooled = sum_ref[...] / jnp.maximum(cnt_ref[...], 1.0).reshape(_NG, 1)
        z = _leaky(jnp.dot(pooled, fc1w_ref[...], precision=_HI,
                           preferred_element_type=jnp.float32) + fc1b_ref[...])
        mu = jnp.mean(z, axis=-1, keepdims=True)
        var = jnp.mean((z - mu) ** 2, axis=-1, keepdims=True)
        z = (z - mu) * jax.lax.rsqrt(var + 1e-5) * lng_ref[...] + lnb_ref[...]
        out_ref[...] = jnp.dot(z, fc2w_ref[...], precision=_HI,
                               preferred_element_type=jnp.float32) + fc2b_ref[...]


def _tc3(acc2, g2, dis, b2, batch_p, fc1_W, fc1_b, ln_g, ln_b, fc2_W, fc2_b):
    return pl.pallas_call(
        _tc3_kernel,
        grid=(_N_PAD // _BLK,),
        in_specs=[
            pl.BlockSpec((2, _BLK, _HD), lambda i: (0, i, 0)),
            pl.BlockSpec((2, _BLK, _HD), lambda i: (0, i, 0)),
            pl.BlockSpec((_BLK, 1), lambda i: (i, 0)),
            pl.BlockSpec((_D,), lambda i: (0,)),
            pl.BlockSpec((_BLK, 1), lambda i: (i, 0)),
            pl.BlockSpec((_D, 128), lambda i: (0, 0)),
            pl.BlockSpec((128,), lambda i: (0,)),
            pl.BlockSpec((128,), lambda i: (0,)),
            pl.BlockSpec((128,), lambda i: (0,)),
            pl.BlockSpec((128, 1), lambda i: (0, 0)),
            pl.BlockSpec((1,), lambda i: (0,)),
        ],
        out_specs=pl.BlockSpec((_NG, 1), lambda i: (0, 0)),
        out_shape=jax.ShapeDtypeStruct((_NG, 1), jnp.float32),
        scratch_shapes=[
            pltpu.VMEM((_NG, _D), jnp.float32),
            pltpu.VMEM((1, _NG), jnp.float32),
        ],
    )(acc2, g2, dis, b2, batch_p, fc1_W, fc1_b, ln_g, ln_b, fc2_W, fc2_b)


def kernel(x, edge_index, batch, W1, b1, W2, b2, fc1_W, fc1_b, ln_g, ln_b,
           fc2_W, fc2_b):
    src, dst = edge_index[0], edge_index[1]
    pad_idx = jnp.full((_E_PAD - _E,), _N, jnp.int32)
    src_p = jnp.concatenate([src, pad_idx])
    dst_p = jnp.concatenate([dst, pad_idx])
    dst_d = dst_p.reshape(32, _NCH_D, 128)
    src_m = src_p.reshape(16, _NCH_M, 128)
    dst_m = dst_p.reshape(16, _NCH_M, 128)
    x_p = jnp.pad(x, ((0, _N_PAD - _N), (0, 0)))
    batch_p = jnp.pad(batch.astype(jnp.int32), (0, _N_PAD - _N),
                      constant_values=_NG).reshape(_N_PAD, 1)
    ones_rows = jnp.ones((128, _HD), jnp.float32)
    zrow = jnp.zeros((128, _HD), jnp.float32)

    deg2 = _sc_deg(dst_d, ones_rows, zrow)
    dis, g1 = _tc1(x_p, W1, deg2)
    acc1 = _sc_msg(g1, src_m, dst_m, zrow)
    g2 = _tc2(acc1, g1, dis, b1, W2)
    acc2 = _sc_msg(g2, src_m, dst_m, zrow)
    return _tc3(acc2, g2, dis, b2, batch_p, fc1_W, fc1_b, ln_g, ln_b,
                fc2_W, fc2_b)
